# initial kernel scaffold (unmeasured)
import jax
import jax.numpy as jnp
from jax import lax
from jax.experimental import pallas as pl
from jax.experimental.pallas import tpu as pltpu


def kernel(
    x,
):
    def body(*refs):
        pass

    out_shape = jax.ShapeDtypeStruct(..., jnp.float32)
    return pl.pallas_call(body, out_shape=out_shape)(...)


# baseline (device time: 31512 ns/iter reference)
import jax
import jax.numpy as jnp
from jax import lax
from jax.experimental import pallas as pl
from jax.experimental.pallas import tpu as pltpu


def kernel(x):
    m, n = x.shape
    half = m // 2

    def body(x_ref, out_ref, recv_buf, sem_s1, sem_r1, sem_s2, sem_r2):
        my_x = lax.axis_index("x")
        my_y = lax.axis_index("y")
        x_nbr = (1 - my_x, my_y)
        y_nbr = (my_x, 1 - my_y)

        barrier_sem = pltpu.get_barrier_semaphore()
        for nbr in (x_nbr, y_nbr):
            pl.semaphore_signal(
                barrier_sem, inc=1,
                device_id=nbr, device_id_type=pl.DeviceIdType.MESH,
            )
        pl.semaphore_wait(barrier_sem, 2)

        row0 = my_y * half

        rdma1 = pltpu.make_async_remote_copy(
            src_ref=x_ref.at[pl.ds(row0, half)],
            dst_ref=recv_buf,
            send_sem=sem_s1,
            recv_sem=sem_r1,
            device_id=x_nbr,
            device_id_type=pl.DeviceIdType.MESH,
        )
        rdma1.start()
        rdma1.wait()
        out_ref[pl.ds(row0, half), :] = (
            x_ref[pl.ds(row0, half), :] + recv_buf[:, :]
        )

        rdma2 = pltpu.make_async_remote_copy(
            src_ref=out_ref.at[pl.ds(row0, half)],
            dst_ref=out_ref.at[pl.ds(row0, half)],
            send_sem=sem_s2,
            recv_sem=sem_r2,
            device_id=y_nbr,
            device_id_type=pl.DeviceIdType.MESH,
        )
        rdma2.start()
        rdma2.wait()

    return pl.pallas_call(
        body,
        out_shape=jax.ShapeDtypeStruct((m, n), x.dtype),
        in_specs=[pl.BlockSpec(memory_space=pltpu.VMEM)],
        out_specs=pl.BlockSpec(memory_space=pltpu.VMEM),
        scratch_shapes=[
            pltpu.VMEM((half, n), x.dtype),
            pltpu.SemaphoreType.DMA,
            pltpu.SemaphoreType.DMA,
            pltpu.SemaphoreType.DMA,
            pltpu.SemaphoreType.DMA,
        ],
        compiler_params=pltpu.CompilerParams(collective_id=0),
    )(x)


# device time: 21845 ns/iter; 1.4425x vs baseline; 1.4425x over previous
import jax
import jax.numpy as jnp
from jax import lax
from jax.experimental import pallas as pl
from jax.experimental.pallas import tpu as pltpu


N_CHUNKS = 8


def kernel(x):
    m, n = x.shape
    half = m // 2
    rows_c = half // N_CHUNKS

    def body(x_ref, out_ref, recv_buf, s1, r1, s2, r2):
        my_x = lax.axis_index("x")
        my_y = lax.axis_index("y")
        x_nbr = (1 - my_x, my_y)
        y_nbr = (my_x, 1 - my_y)

        barrier_sem = pltpu.get_barrier_semaphore()
        for nbr in (x_nbr, y_nbr):
            pl.semaphore_signal(
                barrier_sem, inc=1,
                device_id=nbr, device_id_type=pl.DeviceIdType.MESH,
            )
        pl.semaphore_wait(barrier_sem, 2)

        row0 = my_y * half

        rdma1 = []
        for c in range(N_CHUNKS):
            rd = pltpu.make_async_remote_copy(
                src_ref=x_ref.at[pl.ds(row0 + c * rows_c, rows_c)],
                dst_ref=recv_buf.at[pl.ds(c * rows_c, rows_c)],
                send_sem=s1.at[c],
                recv_sem=r1.at[c],
                device_id=x_nbr,
                device_id_type=pl.DeviceIdType.MESH,
            )
            rd.start()
            rdma1.append(rd)

        rdma2 = []
        for c in range(N_CHUNKS):
            rdma1[c].wait_recv()
            rows = pl.ds(row0 + c * rows_c, rows_c)
            out_ref[rows, :] = (
                x_ref[rows, :] + recv_buf[pl.ds(c * rows_c, rows_c), :]
            )
            rd = pltpu.make_async_remote_copy(
                src_ref=out_ref.at[rows],
                dst_ref=out_ref.at[rows],
                send_sem=s2.at[c],
                recv_sem=r2.at[c],
                device_id=y_nbr,
                device_id_type=pl.DeviceIdType.MESH,
            )
            rd.start()
            rdma2.append(rd)

        for c in range(N_CHUNKS):
            rdma1[c].wait_send()
            rdma2[c].wait()

    return pl.pallas_call(
        body,
        out_shape=jax.ShapeDtypeStruct((m, n), x.dtype),
        in_specs=[pl.BlockSpec(memory_space=pltpu.VMEM)],
        out_specs=pl.BlockSpec(memory_space=pltpu.VMEM),
        scratch_shapes=[
            pltpu.VMEM((half, n), x.dtype),
            pltpu.SemaphoreType.DMA((N_CHUNKS,)),
            pltpu.SemaphoreType.DMA((N_CHUNKS,)),
            pltpu.SemaphoreType.DMA((N_CHUNKS,)),
            pltpu.SemaphoreType.DMA((N_CHUNKS,)),
        ],
        compiler_params=pltpu.CompilerParams(collective_id=0),
    )(x)


# device time: 21424 ns/iter; 1.4709x vs baseline; 1.0197x over previous
import jax
import jax.numpy as jnp
from jax import lax
from jax.experimental import pallas as pl
from jax.experimental.pallas import tpu as pltpu


N_CHUNKS = 16


def kernel(x):
    m, n = x.shape
    half = m // 2
    rows_c = half // N_CHUNKS

    def body(x_ref, out_ref, recv_buf, s1, r1, s2, r2):
        my_x = lax.axis_index("x")
        my_y = lax.axis_index("y")
        x_nbr = (1 - my_x, my_y)
        y_nbr = (my_x, 1 - my_y)

        barrier_sem = pltpu.get_barrier_semaphore()
        for nbr in (x_nbr, y_nbr):
            pl.semaphore_signal(
                barrier_sem, inc=1,
                device_id=nbr, device_id_type=pl.DeviceIdType.MESH,
            )
        pl.semaphore_wait(barrier_sem, 2)

        row0 = my_y * half

        rdma1 = []
        for c in range(N_CHUNKS):
            rd = pltpu.make_async_remote_copy(
                src_ref=x_ref.at[pl.ds(row0 + c * rows_c, rows_c)],
                dst_ref=recv_buf.at[pl.ds(c * rows_c, rows_c)],
                send_sem=s1.at[c],
                recv_sem=r1.at[c],
                device_id=x_nbr,
                device_id_type=pl.DeviceIdType.MESH,
            )
            rd.start()
            rdma1.append(rd)

        rdma2 = []
        for c in range(N_CHUNKS):
            rdma1[c].wait_recv()
            rows = pl.ds(row0 + c * rows_c, rows_c)
            out_ref[rows, :] = (
                x_ref[rows, :] + recv_buf[pl.ds(c * rows_c, rows_c), :]
            )
            rd = pltpu.make_async_remote_copy(
                src_ref=out_ref.at[rows],
                dst_ref=out_ref.at[rows],
                send_sem=s2.at[c],
                recv_sem=r2.at[c],
                device_id=y_nbr,
                device_id_type=pl.DeviceIdType.MESH,
            )
            rd.start()
            rdma2.append(rd)

        for c in range(N_CHUNKS):
            rdma1[c].wait_send()
            rdma2[c].wait()

    return pl.pallas_call(
        body,
        out_shape=jax.ShapeDtypeStruct((m, n), x.dtype),
        in_specs=[pl.BlockSpec(memory_space=pltpu.VMEM)],
        out_specs=pl.BlockSpec(memory_space=pltpu.VMEM),
        scratch_shapes=[
            pltpu.VMEM((half, n), x.dtype),
            pltpu.SemaphoreType.DMA((N_CHUNKS,)),
            pltpu.SemaphoreType.DMA((N_CHUNKS,)),
            pltpu.SemaphoreType.DMA((N_CHUNKS,)),
            pltpu.SemaphoreType.DMA((N_CHUNKS,)),
        ],
        compiler_params=pltpu.CompilerParams(collective_id=0),
    )(x)
